# baseline (speedup 1.0000x reference)
import jax
import jax.numpy as jnp
from jax.experimental import pallas as pl
from jax.experimental.pallas import tpu as pltpu

IN_F = 1024
OUT_F = 4096
BLK = 256
NBLK = OUT_F // BLK
TOPK = NBLK // 4
TOK = 2048
TTILE = 256


def _dense_body(x_ref, wr_ref, br_ref, w1_ref, b1_ref, w2_ref, b2_ref,
                y_ref, mask_ref):
    b = pl.program_id(1)

    @pl.when(b == 0)
    def _router():
        xt = x_ref[...]
        logits = jax.lax.dot_general(
            xt, wr_ref[...], (((1,), (1,)), ((), ())),
            preferred_element_type=jnp.float32) + br_ref[...]
        cols = jax.lax.broadcasted_iota(jnp.int32, (TTILE, NBLK), 1)
        work = logits
        mask = jnp.zeros((TTILE, NBLK), jnp.float32)
        for _ in range(TOPK):
            m = jnp.max(work, axis=1, keepdims=True)
            ismax = work == m
            idxk = jnp.min(jnp.where(ismax, cols, NBLK), axis=1, keepdims=True)
            sel = cols == idxk
            mask = jnp.where(sel, 1.0, mask)
            work = jnp.where(sel, -jnp.inf, work)
        mask_ref[...] = mask

    xt = x_ref[...]
    h = jax.lax.dot_general(xt, w1_ref[...], (((1,), (1,)), ((), ())),
                            preferred_element_type=jnp.float32)
    h = jnp.maximum(h + b1_ref[0], 0.0)
    cols = jax.lax.broadcasted_iota(jnp.int32, (TTILE, NBLK), 1)
    mcol = jnp.max(jnp.where(cols == b, mask_ref[...], 0.0), axis=1,
                   keepdims=True)
    h = h * mcol
    yb = jax.lax.dot_general(h, w2_ref[...], (((1,), (1,)), ((), ())),
                             preferred_element_type=jnp.float32)

    @pl.when(b == 0)
    def _init():
        y_ref[...] = yb + b2_ref[...]

    @pl.when(b != 0)
    def _acc():
        y_ref[...] = y_ref[...] + yb


def kernel(x, Wr, br, W1, b1, W2, b2):
    x2 = x.reshape(TOK, IN_F)
    br2 = br.reshape(1, NBLK)
    b1b = b1.reshape(NBLK, 1, BLK)
    b22 = b2.reshape(1, IN_F)

    grid = (TOK // TTILE, NBLK)
    y = pl.pallas_call(
        _dense_body,
        grid=grid,
        in_specs=[
            pl.BlockSpec((TTILE, IN_F), lambda t, b: (t, 0)),
            pl.BlockSpec((NBLK, IN_F), lambda t, b: (0, 0)),
            pl.BlockSpec((1, NBLK), lambda t, b: (0, 0)),
            pl.BlockSpec((BLK, IN_F), lambda t, b: (b, 0)),
            pl.BlockSpec((1, 1, BLK), lambda t, b: (b, 0, 0)),
            pl.BlockSpec((IN_F, BLK), lambda t, b: (0, b)),
            pl.BlockSpec((1, IN_F), lambda t, b: (0, 0)),
        ],
        out_specs=pl.BlockSpec((TTILE, IN_F), lambda t, b: (t, 0)),
        out_shape=jax.ShapeDtypeStruct((TOK, IN_F), jnp.float32),
        scratch_shapes=[pltpu.VMEM((TTILE, NBLK), jnp.float32)],
    )(x2, Wr, br2, W1, b1b, W2, b22)
    return y.reshape(x.shape)
